# baseline (speedup 1.0000x reference)
import functools

import jax
import jax.numpy as jnp
from jax import lax
from jax.experimental import pallas as pl
from jax.experimental.pallas import tpu as pltpu
from jax.experimental.pallas import tpu_sc as plsc

N = 10000
E = 320000
D = 128
DH = D // 2

NC = 2
NS = 16

EPT = E // NS
C = 512
NCH = 40
EPTP = NCH * C
RPT = N // NS
FB = 125


def _sc_body(inp2_hbm, ei_hbm, x_hbm, ab_hbm, out_hbm, acc_sh,
             ebuf_a, ebuf_b, src_a, src_b, dst_a, dst_b, val_a, val_b,
             rows_a, rows_b, ab_v,
             semi_a, semi_b, semg_a, semg_b, sems_a, sems_b):
    c = lax.axis_index("c")
    s = lax.axis_index("s")

    obuf = rows_a.at[pl.ds(0, FB)]
    xbuf = rows_b.at[pl.ds(0, FB)]

    @plsc.parallel_loop(0, FB, unroll=4)
    def _(r):
        for g in range(DH // 16):
            rows_a[r, pl.ds(g * 16, 16)] = jnp.zeros((16,), jnp.float32)

    def stage_blk(b, carry):
        r0 = s * RPT + b * FB
        pltpu.sync_copy(obuf, acc_sh.at[pl.ds(r0, FB)])
        return carry

    lax.fori_loop(0, RPT // FB, stage_blk, 0)
    plsc.subcore_barrier()

    def start_idx(k, ebuf, sem):
        pltpu.async_copy(ei_hbm.at[s * NCH + k], ebuf, sem)

    def wait_idx(k, ebuf, sem):
        pltpu.make_async_copy(ei_hbm.at[s * NCH + k], ebuf, sem).wait()

    def transform(ebuf, sbuf, dbuf, vbuf):
        @plsc.parallel_loop(0, C // 16, unroll=4)
        def _(g):
            sl = pl.ds(g * 16, 16)
            dbuf[sl] = ebuf[0, sl]
            sbuf[sl] = ebuf[1, sl] + c
            vbuf[sl] = plsc.bitcast(ebuf[2, sl], jnp.float32)

    def start_gather(sbuf, rbuf, sem):
        pltpu.async_copy(inp2_hbm.at[sbuf], rbuf, sem)

    def wait_gather(sbuf, rbuf, sem):
        pltpu.make_async_copy(inp2_hbm.at[sbuf], rbuf, sem).wait()

    def scale(rbuf, vbuf):
        @plsc.parallel_loop(0, C, unroll=8)
        def _(e):
            vs = plsc.load_gather(vbuf, [jnp.full((16,), e, jnp.int32)])
            for g in range(DH // 16):
                rbuf[e, pl.ds(g * 16, 16)] = rbuf[e, pl.ds(g * 16, 16)] * vs

    def start_scatter(rbuf, dbuf, sem):
        pltpu.async_copy(rbuf, acc_sh.at[dbuf], sem, add=True)

    def wait_scatter(rbuf, dbuf, sem):
        pltpu.make_async_copy(rbuf, acc_sh.at[dbuf], sem).wait()

    start_idx(0, ebuf_a, semi_a)
    start_idx(1, ebuf_b, semi_b)
    wait_idx(0, ebuf_a, semi_a)
    transform(ebuf_a, src_a, dst_a, val_a)
    start_idx(2, ebuf_a, semi_a)
    start_gather(src_a, rows_a, semg_a)
    wait_idx(1, ebuf_b, semi_b)
    transform(ebuf_b, src_b, dst_b, val_b)
    start_idx(3, ebuf_b, semi_b)
    start_gather(src_b, rows_b, semg_b)

    def pair_body(g, carry):
        k0 = 2 * g
        wait_gather(src_a, rows_a, semg_a)
        scale(rows_a, val_a)
        start_scatter(rows_a, dst_a, sems_a)
        wait_gather(src_b, rows_b, semg_b)
        scale(rows_b, val_b)
        start_scatter(rows_b, dst_b, sems_b)
        wait_idx(k0 + 2, ebuf_a, semi_a)
        wait_scatter(rows_a, dst_a, sems_a)
        transform(ebuf_a, src_a, dst_a, val_a)

        @pl.when(k0 + 4 <= NCH - 1)
        def _():
            start_idx(k0 + 4, ebuf_a, semi_a)

        start_gather(src_a, rows_a, semg_a)
        wait_idx(k0 + 3, ebuf_b, semi_b)
        wait_scatter(rows_b, dst_b, sems_b)
        transform(ebuf_b, src_b, dst_b, val_b)

        @pl.when(k0 + 5 <= NCH - 1)
        def _():
            start_idx(k0 + 5, ebuf_b, semi_b)

        start_gather(src_b, rows_b, semg_b)
        return carry

    lax.fori_loop(0, NCH // 2 - 1, pair_body, 0)

    wait_gather(src_a, rows_a, semg_a)
    scale(rows_a, val_a)
    start_scatter(rows_a, dst_a, sems_a)
    wait_gather(src_b, rows_b, semg_b)
    scale(rows_b, val_b)
    start_scatter(rows_b, dst_b, sems_b)
    wait_scatter(rows_a, dst_a, sems_a)
    wait_scatter(rows_b, dst_b, sems_b)
    plsc.subcore_barrier()

    pltpu.sync_copy(ab_hbm, ab_v)
    a1 = ab_v[0]
    a2 = ab_v[1]

    def fin_blk(b, carry):
        r0 = s * RPT + b * FB
        pltpu.sync_copy(acc_sh.at[pl.ds(r0, FB)], obuf)
        pltpu.sync_copy(x_hbm.at[pl.ds(r0, FB), pl.ds(c * DH, DH)], xbuf)

        @plsc.parallel_loop(0, FB, unroll=4)
        def _(r):
            for g in range(DH // 16):
                ov = rows_a[r, pl.ds(g * 16, 16)]
                xv = rows_b[r, pl.ds(g * 16, 16)]
                rows_a[r, pl.ds(g * 16, 16)] = a1 * ov + a2 * xv

        pltpu.sync_copy(obuf, out_hbm.at[pl.ds(r0, FB), pl.ds(c * DH, DH)])
        return carry

    lax.fori_loop(0, RPT // FB, fin_blk, 0)


@jax.jit
def _sc_call(inp2, ei, x, ab):
    mesh = plsc.VectorSubcoreMesh(core_axis_name="c", subcore_axis_name="s")
    f = functools.partial(
        pl.kernel,
        out_type=jax.ShapeDtypeStruct((N, D), jnp.float32),
        mesh=mesh,
        compiler_params=pltpu.CompilerParams(
            use_tc_tiling_on_sc=False, needs_layout_passes=False),
        scratch_types=[
            pltpu.VMEM_SHARED((N, DH), jnp.float32),
            pltpu.VMEM((3, C), jnp.int32),
            pltpu.VMEM((3, C), jnp.int32),
            pltpu.VMEM((C,), jnp.int32),
            pltpu.VMEM((C,), jnp.int32),
            pltpu.VMEM((C,), jnp.int32),
            pltpu.VMEM((C,), jnp.int32),
            pltpu.VMEM((C,), jnp.float32),
            pltpu.VMEM((C,), jnp.float32),
            pltpu.VMEM((C, DH), jnp.float32),
            pltpu.VMEM((C, DH), jnp.float32),
            pltpu.VMEM((2, 16), jnp.float32),
            pltpu.SemaphoreType.DMA,
            pltpu.SemaphoreType.DMA,
            pltpu.SemaphoreType.DMA,
            pltpu.SemaphoreType.DMA,
            pltpu.SemaphoreType.DMA,
            pltpu.SemaphoreType.DMA,
        ],
    )(_sc_body)
    return f(inp2, ei, x, ab)


def kernel(inp, adj_indices, adj_values, x, alpha1, alpha2):
    inp2 = inp.reshape(2 * N, DH)
    pad = EPTP - EPT
    padi = jnp.zeros((NS, pad), jnp.int32)
    dst = jnp.concatenate([adj_indices[0].reshape(NS, EPT), padi], axis=1)
    src2 = jnp.concatenate([adj_indices[1].reshape(NS, EPT) * 2, padi], axis=1)
    valb = jnp.concatenate(
        [lax.bitcast_convert_type(adj_values, jnp.int32).reshape(NS, EPT),
         padi], axis=1)
    ei = jnp.stack([dst.reshape(NS, NCH, C), src2.reshape(NS, NCH, C),
                    valb.reshape(NS, NCH, C)], axis=2)
    ei = ei.reshape(NS * NCH, 3, C)
    ab = jnp.stack([jnp.full((16,), alpha1[0], jnp.float32),
                    jnp.full((16,), alpha2[0], jnp.float32)])
    return _sc_call(inp2, ei, x, ab)


# speedup vs baseline: 2.3885x; 2.3885x over previous
import functools

import jax
import jax.numpy as jnp
from jax import lax
from jax.experimental import pallas as pl
from jax.experimental.pallas import tpu as pltpu
from jax.experimental.pallas import tpu_sc as plsc

N = 10000
E = 320000
D = 128
DH = D // 2

NC = 2
NS = 16

EPT = E // NS
C = 400
NCH = 50
EPTP = NCH * C
RPT = N // NS
FB = 125


def _sc_body(inp2_hbm, ei_hbm, x_hbm, ab_hbm, out_hbm, acc_sh,
             ebuf_a, ebuf_b, src_a, src_b, dst_a, dst_b, val_a, val_b,
             rows_a, rows_b, ab_v,
             semi_a, semi_b, semg_a, semg_b, sems_a, sems_b):
    c = lax.axis_index("c")
    s = lax.axis_index("s")

    obuf = rows_a.at[pl.ds(0, FB)]
    xbuf = rows_b.at[pl.ds(0, FB)]

    @plsc.parallel_loop(0, FB, unroll=4)
    def _(r):
        for g in range(DH // 16):
            rows_a[r, pl.ds(g * 16, 16)] = jnp.zeros((16,), jnp.float32)

    def stage_blk(b, carry):
        r0 = s * RPT + b * FB
        pltpu.sync_copy(obuf, acc_sh.at[pl.ds(r0, FB)])
        return carry

    lax.fori_loop(0, RPT // FB, stage_blk, 0)
    plsc.subcore_barrier()

    def start_idx(k, ebuf, sem):
        pltpu.async_copy(ei_hbm.at[s * NCH + k], ebuf, sem)

    def wait_idx(k, ebuf, sem):
        pltpu.make_async_copy(ei_hbm.at[s * NCH + k], ebuf, sem).wait()

    def transform(ebuf, sbuf, dbuf, vbuf):
        @plsc.parallel_loop(0, C // 16, unroll=4)
        def _(g):
            sl = pl.ds(g * 16, 16)
            dbuf[sl] = ebuf[0, sl]
            sbuf[sl] = ebuf[1, sl] + c
            vbuf[sl] = plsc.bitcast(ebuf[2, sl], jnp.float32)

    def start_gather(sbuf, rbuf, sem):
        pltpu.async_copy(inp2_hbm.at[sbuf], rbuf, sem)

    def wait_gather(sbuf, rbuf, sem):
        pltpu.make_async_copy(inp2_hbm.at[sbuf], rbuf, sem).wait()

    def scale(rbuf, vbuf):
        @plsc.parallel_loop(0, C, unroll=8)
        def _(e):
            vs = plsc.load_gather(vbuf, [jnp.full((16,), e, jnp.int32)])
            for g in range(DH // 16):
                rbuf[e, pl.ds(g * 16, 16)] = rbuf[e, pl.ds(g * 16, 16)] * vs

    def start_scatter(rbuf, dbuf, sem):
        pltpu.async_copy(rbuf, acc_sh.at[dbuf], sem, add=True)

    def wait_scatter(rbuf, dbuf, sem):
        pltpu.make_async_copy(rbuf, acc_sh.at[dbuf], sem).wait()

    start_idx(0, ebuf_a, semi_a)
    start_idx(1, ebuf_b, semi_b)
    wait_idx(0, ebuf_a, semi_a)
    transform(ebuf_a, src_a, dst_a, val_a)
    start_idx(2, ebuf_a, semi_a)
    start_gather(src_a, rows_a, semg_a)
    wait_idx(1, ebuf_b, semi_b)
    transform(ebuf_b, src_b, dst_b, val_b)
    start_idx(3, ebuf_b, semi_b)
    start_gather(src_b, rows_b, semg_b)

    def pair_body(g, carry):
        k0 = 2 * g
        wait_gather(src_a, rows_a, semg_a)
        scale(rows_a, val_a)
        start_scatter(rows_a, dst_a, sems_a)
        wait_gather(src_b, rows_b, semg_b)
        scale(rows_b, val_b)
        start_scatter(rows_b, dst_b, sems_b)
        wait_idx(k0 + 2, ebuf_a, semi_a)
        wait_scatter(rows_a, dst_a, sems_a)
        transform(ebuf_a, src_a, dst_a, val_a)

        @pl.when(k0 + 4 <= NCH - 1)
        def _():
            start_idx(k0 + 4, ebuf_a, semi_a)

        start_gather(src_a, rows_a, semg_a)
        wait_idx(k0 + 3, ebuf_b, semi_b)
        wait_scatter(rows_b, dst_b, sems_b)
        transform(ebuf_b, src_b, dst_b, val_b)

        @pl.when(k0 + 5 <= NCH - 1)
        def _():
            start_idx(k0 + 5, ebuf_b, semi_b)

        start_gather(src_b, rows_b, semg_b)
        return carry

    lax.fori_loop(0, NCH // 2 - 1, pair_body, 0)

    wait_gather(src_a, rows_a, semg_a)
    scale(rows_a, val_a)
    start_scatter(rows_a, dst_a, sems_a)
    wait_gather(src_b, rows_b, semg_b)
    scale(rows_b, val_b)
    start_scatter(rows_b, dst_b, sems_b)
    wait_scatter(rows_a, dst_a, sems_a)
    wait_scatter(rows_b, dst_b, sems_b)
    plsc.subcore_barrier()

    pltpu.sync_copy(ab_hbm, ab_v)
    a1 = ab_v[0]
    a2 = ab_v[1]

    def fin_blk(b, carry):
        r0 = s * RPT + b * FB
        pltpu.sync_copy(acc_sh.at[pl.ds(r0, FB)], obuf)
        pltpu.sync_copy(x_hbm.at[pl.ds(r0, FB), pl.ds(c * DH, DH)], xbuf)

        @plsc.parallel_loop(0, FB, unroll=4)
        def _(r):
            for g in range(DH // 16):
                ov = rows_a[r, pl.ds(g * 16, 16)]
                xv = rows_b[r, pl.ds(g * 16, 16)]
                rows_a[r, pl.ds(g * 16, 16)] = a1 * ov + a2 * xv

        pltpu.sync_copy(obuf, out_hbm.at[pl.ds(r0, FB), pl.ds(c * DH, DH)])
        return carry

    lax.fori_loop(0, RPT // FB, fin_blk, 0)


@jax.jit
def _sc_call(inp2, ei, x, ab):
    mesh = plsc.VectorSubcoreMesh(core_axis_name="c", subcore_axis_name="s")
    f = functools.partial(
        pl.kernel,
        out_type=jax.ShapeDtypeStruct((N, D), jnp.float32),
        mesh=mesh,
        compiler_params=pltpu.CompilerParams(
            use_tc_tiling_on_sc=False, needs_layout_passes=False),
        scratch_types=[
            pltpu.VMEM_SHARED((N, DH), jnp.float32),
            pltpu.VMEM((3, C), jnp.int32),
            pltpu.VMEM((3, C), jnp.int32),
            pltpu.VMEM((C,), jnp.int32),
            pltpu.VMEM((C,), jnp.int32),
            pltpu.VMEM((C,), jnp.int32),
            pltpu.VMEM((C,), jnp.int32),
            pltpu.VMEM((C,), jnp.float32),
            pltpu.VMEM((C,), jnp.float32),
            pltpu.VMEM((C, DH), jnp.float32),
            pltpu.VMEM((C, DH), jnp.float32),
            pltpu.VMEM((2, 16), jnp.float32),
            pltpu.SemaphoreType.DMA,
            pltpu.SemaphoreType.DMA,
            pltpu.SemaphoreType.DMA,
            pltpu.SemaphoreType.DMA,
            pltpu.SemaphoreType.DMA,
            pltpu.SemaphoreType.DMA,
        ],
    )(_sc_body)
    return f(inp2, ei, x, ab)


def kernel(inp, adj_indices, adj_values, x, alpha1, alpha2):
    inp2 = inp.reshape(2 * N, DH)
    dst = adj_indices[0]
    src2 = adj_indices[1] * 2
    valb = lax.bitcast_convert_type(adj_values, jnp.int32)
    ei = jnp.stack([dst.reshape(NS, NCH, C), src2.reshape(NS, NCH, C),
                    valb.reshape(NS, NCH, C)], axis=2)
    ei = ei.reshape(NS * NCH, 3, C)
    ab = jnp.stack([jnp.full((16,), alpha1[0], jnp.float32),
                    jnp.full((16,), alpha2[0], jnp.float32)])
    return _sc_call(inp2, ei, x, ab)


# speedup vs baseline: 2.5189x; 1.0546x over previous
import functools

import jax
import jax.numpy as jnp
from jax import lax
from jax.experimental import pallas as pl
from jax.experimental.pallas import tpu as pltpu
from jax.experimental.pallas import tpu_sc as plsc

N = 10000
E = 320000
D = 128
DH = D // 2

NC = 2
NS = 16

EPT = E // NS
C = 400
NCH = 50
EPTP = NCH * C
RPT = N // NS
FB = 125


NB = 3


def _sc_body(inp2_hbm, ei_hbm, x_hbm, ab_hbm, out_hbm, acc_sh,
             ebufs, srcs, dsts, vals, rows, ab_v, semi, semg, sems):
    c = lax.axis_index("c")
    s = lax.axis_index("s")

    rows_a, rows_b = rows[0], rows[1]
    obuf = rows_a.at[pl.ds(0, FB)]
    xbuf = rows_b.at[pl.ds(0, FB)]

    @plsc.parallel_loop(0, FB, unroll=4)
    def _(r):
        for g in range(DH // 16):
            rows_a[r, pl.ds(g * 16, 16)] = jnp.zeros((16,), jnp.float32)

    def stage_blk(b, carry):
        r0 = s * RPT + b * FB
        pltpu.sync_copy(obuf, acc_sh.at[pl.ds(r0, FB)])
        return carry

    lax.fori_loop(0, RPT // FB, stage_blk, 0)
    plsc.subcore_barrier()

    def start_idx(k, j):
        pltpu.async_copy(ei_hbm.at[s * NCH + k], ebufs[j], semi[j])

    def wait_idx(k, j):
        pltpu.make_async_copy(ei_hbm.at[s * NCH + k], ebufs[j], semi[j]).wait()

    def transform(j):
        ebuf, sbuf, dbuf, vbuf = ebufs[j], srcs[j], dsts[j], vals[j]

        @plsc.parallel_loop(0, C // 16, unroll=4)
        def _(g):
            sl = pl.ds(g * 16, 16)
            dbuf[sl] = ebuf[0, sl]
            sbuf[sl] = ebuf[1, sl] + c
            vbuf[sl] = plsc.bitcast(ebuf[2, sl], jnp.float32)

    def start_gather(j):
        pltpu.async_copy(inp2_hbm.at[srcs[j]], rows[j], semg[j])

    def wait_gather(j):
        pltpu.make_async_copy(inp2_hbm.at[srcs[j]], rows[j], semg[j]).wait()

    def scale(j):
        rbuf, vbuf = rows[j], vals[j]

        @plsc.parallel_loop(0, C, unroll=8)
        def _(e):
            vs = plsc.load_gather(vbuf, [jnp.full((16,), e, jnp.int32)])
            for g in range(DH // 16):
                rbuf[e, pl.ds(g * 16, 16)] = rbuf[e, pl.ds(g * 16, 16)] * vs

    def start_scatter(j):
        pltpu.async_copy(rows[j], acc_sh.at[dsts[j]], sems[j], add=True)

    def wait_scatter(j):
        pltpu.make_async_copy(rows[j], acc_sh.at[dsts[j]], sems[j]).wait()

    def body(k, j, first=False, stage=True, prefetch=True):
        jp = (j - 1) % NB
        wait_gather(j)
        scale(j)
        start_scatter(j)
        if not first:
            wait_scatter(jp)
        if stage:
            wait_idx(k + 2, jp)
            transform(jp)
            if prefetch:
                start_idx(k + 5, jp)
            start_gather(jp)

    for j in range(NB):
        start_idx(j, j)
    wait_idx(0, 0)
    transform(0)
    start_idx(3, 0)
    start_gather(0)
    wait_idx(1, 1)
    transform(1)
    start_idx(4, 1)
    start_gather(1)

    body(0, 0, first=True)
    body(1, 1)
    body(2, 2)

    def triple(g, carry):
        k0 = 3 * g
        for j in range(NB):
            k = k0 + j
            jp = (j - 1) % NB
            wait_gather(j)
            scale(j)
            start_scatter(j)
            wait_scatter(jp)
            wait_idx(k + 2, jp)
            transform(jp)

            @pl.when(k <= NCH - 6)
            def _():
                start_idx(k + 5, jp)

            start_gather(jp)
        return carry

    lax.fori_loop(1, (NCH - 5) // 3 + 1, triple, 0)

    body(NCH - 2, (NCH - 2) % NB, stage=False)
    body(NCH - 1, (NCH - 1) % NB, stage=False)
    wait_scatter((NCH - 1) % NB)
    plsc.subcore_barrier()

    pltpu.sync_copy(ab_hbm, ab_v)
    a1 = ab_v[0]
    a2 = ab_v[1]

    def fin_blk(b, carry):
        r0 = s * RPT + b * FB
        pltpu.sync_copy(acc_sh.at[pl.ds(r0, FB)], obuf)
        pltpu.sync_copy(x_hbm.at[pl.ds(r0, FB), pl.ds(c * DH, DH)], xbuf)

        @plsc.parallel_loop(0, FB, unroll=4)
        def _(r):
            for g in range(DH // 16):
                ov = rows_a[r, pl.ds(g * 16, 16)]
                xv = rows_b[r, pl.ds(g * 16, 16)]
                rows_a[r, pl.ds(g * 16, 16)] = a1 * ov + a2 * xv

        pltpu.sync_copy(obuf, out_hbm.at[pl.ds(r0, FB), pl.ds(c * DH, DH)])
        return carry

    lax.fori_loop(0, RPT // FB, fin_blk, 0)


def _sc_body_flat(inp2_hbm, ei_hbm, x_hbm, ab_hbm, out_hbm, acc_sh,
                  eb0, eb1, eb2, sr0, sr1, sr2, ds0, ds1, ds2,
                  va0, va1, va2, ro0, ro1, ro2, ab_v,
                  si0, si1, si2, sg0, sg1, sg2, ss0, ss1, ss2):
    _sc_body(inp2_hbm, ei_hbm, x_hbm, ab_hbm, out_hbm, acc_sh,
             (eb0, eb1, eb2), (sr0, sr1, sr2), (ds0, ds1, ds2),
             (va0, va1, va2), (ro0, ro1, ro2), ab_v,
             (si0, si1, si2), (sg0, sg1, sg2), (ss0, ss1, ss2))


@jax.jit
def _sc_call(inp2, ei, x, ab):
    mesh = plsc.VectorSubcoreMesh(core_axis_name="c", subcore_axis_name="s")
    f = functools.partial(
        pl.kernel,
        out_type=jax.ShapeDtypeStruct((N, D), jnp.float32),
        mesh=mesh,
        compiler_params=pltpu.CompilerParams(
            use_tc_tiling_on_sc=False, needs_layout_passes=False),
        scratch_types=(
            [pltpu.VMEM_SHARED((N, DH), jnp.float32)]
            + [pltpu.VMEM((3, C), jnp.int32)] * NB
            + [pltpu.VMEM((C,), jnp.int32)] * NB
            + [pltpu.VMEM((C,), jnp.int32)] * NB
            + [pltpu.VMEM((C,), jnp.float32)] * NB
            + [pltpu.VMEM((C, DH), jnp.float32)] * NB
            + [pltpu.VMEM((2, 16), jnp.float32)]
            + [pltpu.SemaphoreType.DMA] * (3 * NB)
        ),
    )(_sc_body_flat)
    return f(inp2, ei, x, ab)


def kernel(inp, adj_indices, adj_values, x, alpha1, alpha2):
    inp2 = inp.reshape(2 * N, DH)
    dst = adj_indices[0]
    src2 = adj_indices[1] * 2
    valb = lax.bitcast_convert_type(adj_values, jnp.int32)
    ei = jnp.stack([dst.reshape(NS, NCH, C), src2.reshape(NS, NCH, C),
                    valb.reshape(NS, NCH, C)], axis=2)
    ei = ei.reshape(NS * NCH, 3, C)
    ab = jnp.stack([jnp.full((16,), alpha1[0], jnp.float32),
                    jnp.full((16,), alpha2[0], jnp.float32)])
    return _sc_call(inp2, ei, x, ab)


# speedup vs baseline: 2.5223x; 1.0013x over previous
import functools

import jax
import jax.numpy as jnp
from jax import lax
from jax.experimental import pallas as pl
from jax.experimental.pallas import tpu as pltpu
from jax.experimental.pallas import tpu_sc as plsc

N = 10000
E = 320000
D = 128
DH = D // 2

NC = 2
NS = 16

EPT = E // NS
C = 400
NCH = 50
EPTP = NCH * C
RPT = N // NS
FB = 125


NB = 3


def _sc_body(inp2_hbm, ei_hbm, x_hbm, ab_hbm, out_hbm, acc_sh,
             ebufs, srcs, dsts, vals, rows, ab_v, semi, semg, sems):
    c = lax.axis_index("c")
    s = lax.axis_index("s")

    rows_a, rows_b = rows[0], rows[1]
    obuf = rows_a.at[pl.ds(0, FB)]
    xbuf = rows_b.at[pl.ds(0, FB)]

    @plsc.parallel_loop(0, FB, unroll=4)
    def _(r):
        for g in range(DH // 16):
            rows_a[r, pl.ds(g * 16, 16)] = jnp.zeros((16,), jnp.float32)

    def stage_blk(b, carry):
        r0 = s * RPT + b * FB
        pltpu.sync_copy(obuf, acc_sh.at[pl.ds(r0, FB)])
        return carry

    lax.fori_loop(0, RPT // FB, stage_blk, 0)
    plsc.subcore_barrier()

    def start_idx(k, j):
        pltpu.async_copy(ei_hbm.at[s * NCH + k], ebufs[j], semi[j])

    def wait_idx(k, j):
        pltpu.make_async_copy(ei_hbm.at[s * NCH + k], ebufs[j], semi[j]).wait()

    def transform(j):
        ebuf, sbuf, dbuf, vbuf = ebufs[j], srcs[j], dsts[j], vals[j]

        @plsc.parallel_loop(0, C // 16, unroll=8)
        def _(g):
            sl = pl.ds(g * 16, 16)
            dbuf[sl] = ebuf[0, sl]
            sbuf[sl] = ebuf[1, sl] + c
            vbuf[sl] = plsc.bitcast(ebuf[2, sl], jnp.float32)

    def start_gather(j):
        pltpu.async_copy(inp2_hbm.at[srcs[j]], rows[j], semg[j])

    def wait_gather(j):
        pltpu.make_async_copy(inp2_hbm.at[srcs[j]], rows[j], semg[j]).wait()

    def scale(j):
        rbuf, vbuf = rows[j], vals[j]

        @plsc.parallel_loop(0, C, unroll=16)
        def _(e):
            vs = plsc.load_gather(vbuf, [jnp.full((16,), e, jnp.int32)])
            for g in range(DH // 16):
                rbuf[e, pl.ds(g * 16, 16)] = rbuf[e, pl.ds(g * 16, 16)] * vs

    def start_scatter(j):
        pltpu.async_copy(rows[j], acc_sh.at[dsts[j]], sems[j], add=True)

    def wait_scatter(j):
        pltpu.make_async_copy(rows[j], acc_sh.at[dsts[j]], sems[j]).wait()

    def body(k, j, first=False, stage=True, prefetch=True):
        jp = (j - 1) % NB
        wait_gather(j)
        scale(j)
        start_scatter(j)
        if not first:
            wait_scatter(jp)
        if stage:
            wait_idx(k + 2, jp)
            transform(jp)
            if prefetch:
                start_idx(k + 5, jp)
            start_gather(jp)

    for j in range(NB):
        start_idx(j, j)
    wait_idx(0, 0)
    transform(0)
    start_idx(3, 0)
    start_gather(0)
    wait_idx(1, 1)
    transform(1)
    start_idx(4, 1)
    start_gather(1)

    body(0, 0, first=True)
    body(1, 1)
    body(2, 2)

    def triple(g, carry):
        k0 = 3 * g
        for j in range(NB):
            k = k0 + j
            jp = (j - 1) % NB
            wait_gather(j)
            scale(j)
            start_scatter(j)
            wait_scatter(jp)
            wait_idx(k + 2, jp)
            transform(jp)

            @pl.when(k <= NCH - 6)
            def _():
                start_idx(k + 5, jp)

            start_gather(jp)
        return carry

    lax.fori_loop(1, (NCH - 5) // 3 + 1, triple, 0)

    body(NCH - 2, (NCH - 2) % NB, stage=False)
    body(NCH - 1, (NCH - 1) % NB, stage=False)
    wait_scatter((NCH - 1) % NB)
    plsc.subcore_barrier()

    pltpu.sync_copy(ab_hbm, ab_v)
    a1 = ab_v[0]
    a2 = ab_v[1]

    def fin_blk(b, carry):
        r0 = s * RPT + b * FB
        pltpu.sync_copy(acc_sh.at[pl.ds(r0, FB)], obuf)
        pltpu.sync_copy(x_hbm.at[pl.ds(r0, FB), pl.ds(c * DH, DH)], xbuf)

        @plsc.parallel_loop(0, FB, unroll=4)
        def _(r):
            for g in range(DH // 16):
                ov = rows_a[r, pl.ds(g * 16, 16)]
                xv = rows_b[r, pl.ds(g * 16, 16)]
                rows_a[r, pl.ds(g * 16, 16)] = a1 * ov + a2 * xv

        pltpu.sync_copy(obuf, out_hbm.at[pl.ds(r0, FB), pl.ds(c * DH, DH)])
        return carry

    lax.fori_loop(0, RPT // FB, fin_blk, 0)


def _sc_body_flat(inp2_hbm, ei_hbm, x_hbm, ab_hbm, out_hbm, acc_sh,
                  eb0, eb1, eb2, sr0, sr1, sr2, ds0, ds1, ds2,
                  va0, va1, va2, ro0, ro1, ro2, ab_v,
                  si0, si1, si2, sg0, sg1, sg2, ss0, ss1, ss2):
    _sc_body(inp2_hbm, ei_hbm, x_hbm, ab_hbm, out_hbm, acc_sh,
             (eb0, eb1, eb2), (sr0, sr1, sr2), (ds0, ds1, ds2),
             (va0, va1, va2), (ro0, ro1, ro2), ab_v,
             (si0, si1, si2), (sg0, sg1, sg2), (ss0, ss1, ss2))


@jax.jit
def _sc_call(inp2, ei, x, ab):
    mesh = plsc.VectorSubcoreMesh(core_axis_name="c", subcore_axis_name="s")
    f = functools.partial(
        pl.kernel,
        out_type=jax.ShapeDtypeStruct((N, D), jnp.float32),
        mesh=mesh,
        compiler_params=pltpu.CompilerParams(
            use_tc_tiling_on_sc=False, needs_layout_passes=False),
        scratch_types=(
            [pltpu.VMEM_SHARED((N, DH), jnp.float32)]
            + [pltpu.VMEM((3, C), jnp.int32)] * NB
            + [pltpu.VMEM((C,), jnp.int32)] * NB
            + [pltpu.VMEM((C,), jnp.int32)] * NB
            + [pltpu.VMEM((C,), jnp.float32)] * NB
            + [pltpu.VMEM((C, DH), jnp.float32)] * NB
            + [pltpu.VMEM((2, 16), jnp.float32)]
            + [pltpu.SemaphoreType.DMA] * (3 * NB)
        ),
    )(_sc_body_flat)
    return f(inp2, ei, x, ab)


def kernel(inp, adj_indices, adj_values, x, alpha1, alpha2):
    inp2 = inp.reshape(2 * N, DH)
    dst = adj_indices[0]
    src2 = adj_indices[1] * 2
    valb = lax.bitcast_convert_type(adj_values, jnp.int32)
    ei = jnp.stack([dst.reshape(NS, NCH, C), src2.reshape(NS, NCH, C),
                    valb.reshape(NS, NCH, C)], axis=2)
    ei = ei.reshape(NS * NCH, 3, C)
    ab = jnp.stack([jnp.full((16,), alpha1[0], jnp.float32),
                    jnp.full((16,), alpha2[0], jnp.float32)])
    return _sc_call(inp2, ei, x, ab)
